# Initial kernel scaffold; baseline (speedup 1.0000x reference)
import jax
import jax.numpy as jnp
from jax.experimental import pallas as pl


def kernel(x, edge_index, W_bef, b_bef, W_aft, b_aft):
    raise NotImplementedError("write your pallas kernel here")


# speedup vs baseline: 6.6240x; 6.6240x over previous
import functools

import jax
import jax.numpy as jnp
from jax import lax
from jax.experimental import pallas as pl
from jax.experimental.pallas import tpu as pltpu
from jax.experimental.pallas import tpu_sc as plsc

N = 10000
E = 320000
D_IN = 128
D_HID = 128
D_OUT = 64
PROP = 8
ALPHA = 0.5

NC = 2
NS = 16
NW = NC * NS
LANE = 128

NPAD = 10240
RPS = NPAD // NS
EPT = 10112
NROW = EPT // LANE
EPAD = EPT * NW

BN = 2048
GRID = NPAD // BN

_mesh = plsc.VectorSubcoreMesh(core_axis_name="c", subcore_axis_name="s")


def _mlp_bef_body(x_ref, w_ref, b_ref, o_ref):
    i = pl.program_id(0)
    h = jnp.maximum(jnp.dot(x_ref[...], w_ref[...],
                            preferred_element_type=jnp.float32) + b_ref[...],
                    0.0)
    row = i * BN + lax.broadcasted_iota(jnp.int32, (BN, 1), 0)
    o_ref[...] = jnp.where(row < N, h, 0.0)


def _mlp_bef(xp, w, b):
    return pl.pallas_call(
        _mlp_bef_body,
        grid=(GRID,),
        in_specs=[
            pl.BlockSpec((BN, D_IN), lambda i: (i, 0)),
            pl.BlockSpec((D_IN, D_HID), lambda i: (0, 0)),
            pl.BlockSpec((1, D_HID), lambda i: (0, 0)),
        ],
        out_specs=pl.BlockSpec((BN, D_HID), lambda i: (i, 0)),
        out_shape=jax.ShapeDtypeStruct((NPAD, D_HID), jnp.float32),
    )(xp, w, b)


def _finalize_body(degp_ref, h0_ref, norm_ref, norm2_ref, g0_ref):
    deg = degp_ref[0, :] + degp_ref[1, :]
    nrm = lax.rsqrt(jnp.clip(deg, 1.0, None))
    ncol = jnp.reshape(nrm, (NPAD, 1))
    norm_ref[...] = ncol
    norm2_ref[...] = ncol * ncol
    g0_ref[...] = ncol * h0_ref[...]


def _finalize(degp, h0p):
    return pl.pallas_call(
        _finalize_body,
        out_shape=(
            jax.ShapeDtypeStruct((NPAD, 1), jnp.float32),
            jax.ShapeDtypeStruct((NPAD, 1), jnp.float32),
            jax.ShapeDtypeStruct((NPAD, D_HID), jnp.float32),
        ),
    )(degp, h0p)


def _combine_body(aggp_ref, n2_ref, g0_ref, u_ref):
    s = aggp_ref[0] + aggp_ref[1]
    u_ref[...] = (1.0 - ALPHA) * n2_ref[...] * s + ALPHA * g0_ref[...]


def _combine(aggp, norm2c, g0):
    return pl.pallas_call(
        _combine_body,
        grid=(GRID,),
        in_specs=[
            pl.BlockSpec((2, BN, D_HID), lambda i: (0, i, 0)),
            pl.BlockSpec((BN, 1), lambda i: (i, 0)),
            pl.BlockSpec((BN, D_HID), lambda i: (i, 0)),
        ],
        out_specs=pl.BlockSpec((BN, D_HID), lambda i: (i, 0)),
        out_shape=jax.ShapeDtypeStruct((NPAD, D_HID), jnp.float32),
    )(aggp, norm2c, g0)


def _mlp_aft_body(aggp_ref, n_ref, h0_ref, w_ref, b_ref, o_ref):
    s = aggp_ref[0] + aggp_ref[1]
    h = (1.0 - ALPHA) * n_ref[...] * s + ALPHA * h0_ref[...]
    o_ref[...] = jnp.dot(h, w_ref[...],
                         preferred_element_type=jnp.float32) + b_ref[...]


def _mlp_aft(aggp, normc, h0p, w, b):
    return pl.pallas_call(
        _mlp_aft_body,
        grid=(GRID,),
        in_specs=[
            pl.BlockSpec((2, BN, D_HID), lambda i: (0, i, 0)),
            pl.BlockSpec((BN, 1), lambda i: (i, 0)),
            pl.BlockSpec((BN, D_HID), lambda i: (i, 0)),
            pl.BlockSpec((D_HID, D_OUT), lambda i: (0, 0)),
            pl.BlockSpec((1, D_OUT), lambda i: (0, 0)),
        ],
        out_specs=pl.BlockSpec((BN, D_OUT), lambda i: (i, 0)),
        out_shape=jax.ShapeDtypeStruct((NPAD, D_OUT), jnp.float32),
    )(aggp, normc, h0p, w, b)


def _deg_body(src_hbm, dst_hbm, zeros1_hbm, degp_hbm,
              ones_v, idxs_v, idxd_v, deg_sh):
    c = lax.axis_index("c")
    s = lax.axis_index("s")
    wid = c * NS + s
    for i in range(LANE // 16):
        ones_v[pl.ds(16 * i, 16)] = jnp.full((16,), 1.0, jnp.float32)
    pltpu.sync_copy(zeros1_hbm.at[pl.ds(s * RPS, RPS)],
                    deg_sh.at[pl.ds(s * RPS, RPS)])
    plsc.subcore_barrier()
    pltpu.sync_copy(src_hbm.at[wid], idxs_v)
    pltpu.sync_copy(dst_hbm.at[wid], idxd_v)

    def body(j, carry):
        pltpu.sync_copy(ones_v, deg_sh.at[idxs_v.at[j]], add=True)
        pltpu.sync_copy(ones_v, deg_sh.at[idxd_v.at[j]], add=True)
        return carry

    lax.fori_loop(0, NROW, body, 0)
    plsc.subcore_barrier()
    pltpu.sync_copy(deg_sh.at[pl.ds(s * RPS, RPS)],
                    degp_hbm.at[c, pl.ds(s * RPS, RPS)])


_deg_call = pl.kernel(
    _deg_body,
    out_type=jax.ShapeDtypeStruct((NC, NPAD), jnp.float32),
    mesh=_mesh,
    scratch_types=[
        pltpu.VMEM((LANE,), jnp.float32),
        pltpu.VMEM((NROW, LANE), jnp.int32),
        pltpu.VMEM((NROW, LANE), jnp.int32),
        pltpu.VMEM_SHARED((NPAD,), jnp.float32),
    ],
)


def _round_body(u_hbm, src_hbm, dst_hbm, zeros2_hbm, aggp_hbm,
                idxs_v, idxd_v, rows_v, agg_sh, sem):
    c = lax.axis_index("c")
    s = lax.axis_index("s")
    wid = c * NS + s
    pltpu.sync_copy(zeros2_hbm.at[pl.ds(s * RPS, RPS)],
                    agg_sh.at[pl.ds(s * RPS, RPS)])
    plsc.subcore_barrier()
    pltpu.sync_copy(src_hbm.at[wid], idxs_v)
    pltpu.sync_copy(dst_hbm.at[wid], idxd_v)

    def body(j, carry):
        pltpu.async_copy(u_hbm.at[idxs_v.at[j]], rows_v, sem).wait()
        pltpu.sync_copy(rows_v, agg_sh.at[idxd_v.at[j]], add=True)
        return carry

    lax.fori_loop(0, NROW, body, 0)
    plsc.subcore_barrier()
    pltpu.sync_copy(agg_sh.at[pl.ds(s * RPS, RPS)],
                    aggp_hbm.at[c, pl.ds(s * RPS, RPS)])


_round_call = pl.kernel(
    _round_body,
    out_type=jax.ShapeDtypeStruct((NC, NPAD, D_HID), jnp.float32),
    mesh=_mesh,
    scratch_types=[
        pltpu.VMEM((NROW, LANE), jnp.int32),
        pltpu.VMEM((NROW, LANE), jnp.int32),
        pltpu.VMEM((LANE, D_HID), jnp.float32),
        pltpu.VMEM_SHARED((NPAD, D_HID), jnp.float32),
        pltpu.SemaphoreType.DMA,
    ],
)


@jax.jit
def kernel(x, edge_index, W_bef, b_bef, W_aft, b_aft):
    src = edge_index[0].astype(jnp.int32)
    dst = edge_index[1].astype(jnp.int32)
    pad = EPAD - E
    fill = jnp.full((pad,), N, jnp.int32)
    srcp = jnp.concatenate([src, fill]).reshape(NW, NROW, LANE)
    dstp = jnp.concatenate([dst, fill]).reshape(NW, NROW, LANE)
    xp = jnp.pad(x, ((0, NPAD - N), (0, 0)))
    zeros1 = jnp.zeros((NPAD,), jnp.float32)
    zeros2 = jnp.zeros((NPAD, D_HID), jnp.float32)

    h0p = _mlp_bef(xp, W_bef, b_bef.reshape(1, D_HID))
    degp = _deg_call(srcp, dstp, zeros1)
    normc, norm2c, g0 = _finalize(degp, h0p)

    u = g0
    for _ in range(PROP - 1):
        aggp = _round_call(u, srcp, dstp, zeros2)
        u = _combine(aggp, norm2c, g0)
    aggp = _round_call(u, srcp, dstp, zeros2)
    outp = _mlp_aft(aggp, normc, h0p, W_aft, b_aft.reshape(1, D_OUT))
    return outp[:N]
